# baseline (speedup 1.0000x reference)
import functools

import jax
import jax.numpy as jnp
from jax import lax
from jax.experimental import pallas as pl
from jax.experimental.pallas import tpu as pltpu
from jax.experimental.pallas import tpu_sc as plsc

NUSER = 1000000
NITEM = 100000
HID = 64
BATCH = 4096

_info = plsc.get_sparse_core_info()
_NC, _NS, _NL = _info.num_cores, _info.num_subcores, _info.num_lanes
_NW = _NC * _NS
_BPW = BATCH // _NW
_NG = _BPW // _NL


def _half_select(wide_v, colbase_v, out_v):
    lanes = lax.iota(jnp.int32, _NL)

    def body(g):
        row_vec = g * _NL + lanes
        col0 = colbase_v[pl.ds(g * _NL, _NL)]
        for c in range(HID):
            val = plsc.load_gather(wide_v, [row_vec, col0 + c])
            plsc.store_scatter(out_v, [row_vec, jnp.full((_NL,), c, jnp.int32)], val)

    pl.loop(0, _NG)(body)


@functools.partial(
    pl.kernel,
    mesh=plsc.VectorSubcoreMesh(core_axis_name="c", subcore_axis_name="s"),
    out_type=[
        jax.ShapeDtypeStruct((BATCH, HID), jnp.float32),
        jax.ShapeDtypeStruct((BATCH, HID), jnp.float32),
    ],
    scratch_types=[
        pltpu.VMEM((_BPW,), jnp.int32),
        pltpu.VMEM((_BPW,), jnp.int32),
        pltpu.VMEM((_BPW, 2 * HID), jnp.float32),
        pltpu.VMEM((_BPW, HID), jnp.float32),
        pltpu.VMEM((_BPW,), jnp.int32),
        pltpu.VMEM((_BPW,), jnp.int32),
        pltpu.VMEM((_BPW, 2 * HID), jnp.float32),
        pltpu.VMEM((_BPW, HID), jnp.float32),
        pltpu.SemaphoreType.DMA,
        pltpu.SemaphoreType.DMA,
    ],
    compiler_params=pltpu.CompilerParams(needs_layout_passes=False),
)
def _lookup(upair_hbm, ucol_hbm, ipair_hbm, icol_hbm, uf_hbm, if_hbm,
            uout_hbm, iout_hbm,
            upair_v, ucol_v, uwide_v, uout_v,
            ipair_v, icol_v, iwide_v, iout_v, usem, isem):
    wid = lax.axis_index("s") * _NC + lax.axis_index("c")
    base = wid * _BPW
    pltpu.sync_copy(upair_hbm.at[pl.ds(base, _BPW)], upair_v)
    pltpu.sync_copy(ipair_hbm.at[pl.ds(base, _BPW)], ipair_v)
    pltpu.sync_copy(ucol_hbm.at[pl.ds(base, _BPW)], ucol_v)
    pltpu.sync_copy(icol_hbm.at[pl.ds(base, _BPW)], icol_v)
    ucopy = pltpu.async_copy(uf_hbm.at[upair_v], uwide_v, usem)
    icopy = pltpu.async_copy(if_hbm.at[ipair_v], iwide_v, isem)
    ucopy.wait()
    _half_select(uwide_v, ucol_v, uout_v)
    uw = pltpu.async_copy(uout_v, uout_hbm.at[pl.ds(base, _BPW)], usem)
    icopy.wait()
    _half_select(iwide_v, icol_v, iout_v)
    iw = pltpu.async_copy(iout_v, iout_hbm.at[pl.ds(base, _BPW)], isem)
    uw.wait()
    iw.wait()


def kernel(user, item, user_factor, item_factor):
    user = user.astype(jnp.int32)
    item = item.astype(jnp.int32)
    upair, ucol = user >> 1, (user & 1) * HID
    ipair, icol = item >> 1, (item & 1) * HID
    uf2 = user_factor.reshape(NUSER // 2, 2 * HID)
    if2 = item_factor.reshape(NITEM // 2, 2 * HID)
    user_emb, item_emb = _lookup(upair, ucol, ipair, icol, uf2, if2)
    return (user_emb, item_emb)


# speedup vs baseline: 1.7077x; 1.7077x over previous
import functools

import jax
import jax.numpy as jnp
from jax import lax
from jax.experimental import pallas as pl
from jax.experimental.pallas import tpu as pltpu
from jax.experimental.pallas import tpu_sc as plsc

NUSER = 1000000
NITEM = 100000
HID = 64
BATCH = 4096

_info = plsc.get_sparse_core_info()
_NC, _NS = _info.num_cores, _info.num_subcores
_NW = _NC * _NS
_BPW = BATCH // _NW


@functools.partial(
    pl.kernel,
    mesh=plsc.VectorSubcoreMesh(core_axis_name="c", subcore_axis_name="s"),
    out_type=[
        jax.ShapeDtypeStruct((BATCH, HID), jnp.float32),
        jax.ShapeDtypeStruct((BATCH, HID), jnp.float32),
    ],
    scratch_types=[
        pltpu.VMEM((_BPW,), jnp.int32),
        pltpu.VMEM((_BPW, HID), jnp.float32),
        pltpu.VMEM((_BPW,), jnp.int32),
        pltpu.VMEM((_BPW, HID), jnp.float32),
        pltpu.SemaphoreType.DMA,
        pltpu.SemaphoreType.DMA,
    ],
)
def _lookup(user_hbm, item_hbm, uf_hbm, if_hbm, uout_hbm, iout_hbm,
            uidx_v, urows_v, iidx_v, irows_v, usem, isem):
    wid = lax.axis_index("s") * _NC + lax.axis_index("c")
    base = wid * _BPW
    pltpu.sync_copy(user_hbm.at[pl.ds(base, _BPW)], uidx_v)
    pltpu.sync_copy(item_hbm.at[pl.ds(base, _BPW)], iidx_v)

    def fire(g):
        uvec = uidx_v[pl.ds(g * 16, 16)]
        ivec = iidx_v[pl.ds(g * 16, 16)]
        for j in range(16):
            r = g * 16 + j
            pltpu.async_copy(uf_hbm.at[uvec[j]], urows_v.at[r], usem)
            pltpu.async_copy(if_hbm.at[ivec[j]], irows_v.at[r], isem)

    pl.loop(0, _BPW // 16)(fire)
    pltpu.make_async_copy(uf_hbm.at[pl.ds(0, _BPW)], urows_v, usem).wait()
    pltpu.make_async_copy(if_hbm.at[pl.ds(0, _BPW)], irows_v, isem).wait()
    uw = pltpu.async_copy(urows_v, uout_hbm.at[pl.ds(base, _BPW)], usem)
    iw = pltpu.async_copy(irows_v, iout_hbm.at[pl.ds(base, _BPW)], isem)
    uw.wait()
    iw.wait()


def kernel(user, item, user_factor, item_factor):
    user = user.astype(jnp.int32)
    item = item.astype(jnp.int32)
    user_emb, item_emb = _lookup(user, item, user_factor, item_factor)
    return (user_emb, item_emb)


# speedup vs baseline: 4.8429x; 2.8359x over previous
import functools

import jax
import jax.numpy as jnp
from jax import lax
from jax.experimental import pallas as pl
from jax.experimental.pallas import tpu as pltpu
from jax.experimental.pallas import tpu_sc as plsc

NUSER = 1000000
NITEM = 100000
HID = 64
BATCH = 4096

_info = plsc.get_sparse_core_info()
_NC, _NS, _NL = _info.num_cores, _info.num_subcores, _info.num_lanes
_NW = _NC * _NS
_BPW = BATCH // _NW
_NBUF = 4


@functools.partial(
    pl.kernel,
    mesh=plsc.VectorSubcoreMesh(core_axis_name="c", subcore_axis_name="s"),
    out_type=[
        jax.ShapeDtypeStruct((HID, BATCH), jnp.float32),
        jax.ShapeDtypeStruct((HID, BATCH), jnp.float32),
    ],
    scratch_types=(
        [pltpu.VMEM((_BPW,), jnp.int32)] * 2
        + [pltpu.VMEM((HID, 128), jnp.float32)] * (2 * _NBUF)
        + [pltpu.VMEM((HID, _BPW), jnp.float32)] * 2
        + [pltpu.SemaphoreType.DMA] * 2
    ),
    compiler_params=pltpu.CompilerParams(
        needs_layout_passes=False, disable_bounds_checks=True
    ),
)
def _lookup(user_hbm, item_hbm, uft_hbm, ift_hbm, uout_hbm, iout_hbm,
            uidx_v, iidx_v,
            ublk0, ublk1, ublk2, ublk3, iblk0, iblk1, iblk2, iblk3,
            ucols_v, icols_v, usem, isem):
    ublks = (ublk0, ublk1, ublk2, ublk3)
    iblks = (iblk0, iblk1, iblk2, iblk3)
    wid = lax.axis_index("s") * _NC + lax.axis_index("c")
    base = pl.multiple_of(wid * _BPW, _BPW)
    pltpu.sync_copy(user_hbm.at[pl.ds(base, _BPW)], uidx_v)
    pltpu.sync_copy(item_hbm.at[pl.ds(base, _BPW)], iidx_v)
    lanes = lax.iota(jnp.int32, _NL)

    def extract(blk, cols, rr, r):
        rr_v = jnp.full((_NL,), rr, jnp.int32)
        r_v = jnp.full((_NL,), r, jnp.int32)
        for k in range(HID // _NL):
            cvec = k * _NL + lanes
            val = plsc.load_gather(blk, [cvec, rr_v])
            plsc.store_scatter(cols, [cvec, r_v], val)

    def group(g):
        uvec = uidx_v[pl.ds(g * 16, 16)]
        ivec = iidx_v[pl.ds(g * 16, 16)]
        for h in range(16 // _NBUF):
            uhandles, ihandles = [], []
            for b in range(_NBUF):
                j = h * _NBUF + b
                ur0 = pl.multiple_of((uvec[j] >> 7) * 128, 128)
                ir0 = pl.multiple_of((ivec[j] >> 7) * 128, 128)
                uhandles.append(pltpu.async_copy(
                    uft_hbm.at[:, pl.ds(ur0, 128)], ublks[b], usem))
                ihandles.append(pltpu.async_copy(
                    ift_hbm.at[:, pl.ds(ir0, 128)], iblks[b], isem))
            for b in range(_NBUF):
                j = h * _NBUF + b
                r = g * 16 + j
                uhandles[b].wait()
                extract(ublks[b], ucols_v, uvec[j] & 127, r)
                ihandles[b].wait()
                extract(iblks[b], icols_v, ivec[j] & 127, r)

    pl.loop(0, _BPW // 16)(group)
    uw = pltpu.async_copy(ucols_v, uout_hbm.at[:, pl.ds(base, _BPW)], usem)
    iw = pltpu.async_copy(icols_v, iout_hbm.at[:, pl.ds(base, _BPW)], isem)
    uw.wait()
    iw.wait()


def kernel(user, item, user_factor, item_factor):
    user = user.astype(jnp.int32)
    item = item.astype(jnp.int32)
    uout_t, iout_t = _lookup(user, item, user_factor.T, item_factor.T)
    return (uout_t.T, iout_t.T)
